# baseline (speedup 1.0000x reference)
import functools

import jax
import jax.numpy as jnp
from jax import lax
from jax.experimental import pallas as pl
from jax.experimental.pallas import tpu as pltpu
from jax.experimental.pallas import tpu_sc as plsc

_NUM_CODES = 8192
_CODE_DIM = 256
_BETA = 0.25

_ROW_BLOCK = 256
_GROUPS = ((0, 2816), (2816, 5632), (5632, 8192))


def _dist_argmin_body(z_ref, cb_ref, rn_ref, cn_ref, idx_ref, acc_ref):
    i = pl.program_id(0)
    zm2 = (z_ref[...] * (-2.0)).astype(jnp.bfloat16)
    rn = rn_ref[...]
    lane = lax.broadcasted_iota(jnp.int32, (_ROW_BLOCK, 128), 1)

    run_cmp = None
    run_val = None
    run_idx = None
    for g, (c0, c1) in enumerate(_GROUPS):
        dotm = jnp.dot(zm2, cb_ref[:, c0:c1],
                       preferred_element_type=jnp.float32)
        val = (rn + dotm[:, 0:128]) + cn_ref[:, c0:c0 + 128]
        kidx = jnp.zeros((_ROW_BLOCK, 128), jnp.int32)
        for k in range(1, (c1 - c0) // 128):
            dk = ((rn + dotm[:, k * 128:(k + 1) * 128])
                  + cn_ref[:, c0 + k * 128:c0 + (k + 1) * 128])
            hit = dk < val
            val = jnp.minimum(val, dk)
            kidx = jnp.where(hit, k, kidx)
        mv = jnp.min(val, axis=1, keepdims=True)
        gidx = kidx * 128 + lane + c0
        mi = jnp.min(jnp.where(val == mv, gidx, _NUM_CODES),
                     axis=1, keepdims=True)
        if g == 0:
            run_cmp, run_val, run_idx = mv, mv, mi
        else:
            better = mv < run_cmp
            run_cmp = jnp.where(better, mv, run_cmp)
            run_val = jnp.where(better, mv, run_val)
            run_idx = jnp.where(better, mi, run_idx)
        if g != len(_GROUPS) - 1:
            run_cmp = run_cmp.astype(jnp.bfloat16).astype(jnp.float32)

    idx_ref[...] = run_idx

    @pl.when(i == 0)
    def _():
        acc_ref[...] = jnp.zeros((1, 1), jnp.float32)

    acc_ref[...] += jnp.sum(run_val, keepdims=True)


def _sc_gather(table, idx):
    nc, ns = 2, 16
    nw = nc * ns
    b = idx.shape[0]
    d_ = table.shape[1]
    per_w = b // nw
    ch = 256
    n_ch = per_w // ch
    mesh = plsc.VectorSubcoreMesh(core_axis_name="c", subcore_axis_name="s")

    @functools.partial(
        pl.kernel, mesh=mesh,
        out_type=jax.ShapeDtypeStruct((b, d_), jnp.float32),
        scratch_types=[
            pltpu.VMEM((ch,), jnp.int32),
            pltpu.VMEM((ch, d_), jnp.float32),
            pltpu.SemaphoreType.DMA,
        ],
    )
    def k(table_hbm, idx_hbm, out_hbm, idx_v, rows_v, sem):
        wid = lax.axis_index("s") * nc + lax.axis_index("c")
        for c in range(n_ch):
            base = wid * per_w + c * ch
            pltpu.sync_copy(idx_hbm.at[pl.ds(base, ch)], idx_v)
            pltpu.async_copy(table_hbm.at[idx_v], rows_v, sem).wait()
            pltpu.sync_copy(rows_v, out_hbm.at[pl.ds(base, ch)])

    return k(table, idx)


def kernel(z, codebook):
    z_flat = z.reshape(-1, _CODE_DIM)
    n_rows = z_flat.shape[0]
    rn = jnp.sum(z_flat ** 2, axis=1, keepdims=True)
    cn = jnp.sum(codebook ** 2, axis=0, keepdims=True)

    grid = n_rows // _ROW_BLOCK
    idx2, acc = pl.pallas_call(
        _dist_argmin_body,
        grid=(grid,),
        in_specs=[
            pl.BlockSpec((_ROW_BLOCK, _CODE_DIM), lambda i: (i, 0)),
            pl.BlockSpec((_CODE_DIM, _NUM_CODES), lambda i: (0, 0)),
            pl.BlockSpec((_ROW_BLOCK, 1), lambda i: (i, 0)),
            pl.BlockSpec((1, _NUM_CODES), lambda i: (0, 0)),
        ],
        out_specs=[
            pl.BlockSpec((_ROW_BLOCK, 1), lambda i: (i, 0)),
            pl.BlockSpec((1, 1), lambda i: (0, 0)),
        ],
        out_shape=[
            jax.ShapeDtypeStruct((n_rows, 1), jnp.int32),
            jax.ShapeDtypeStruct((1, 1), jnp.float32),
        ],
    )(z_flat, codebook.astype(jnp.bfloat16), rn, cn)

    indices = idx2.reshape(z.shape[:-1])
    loss = acc[0, 0] * ((1.0 + _BETA) / z.size)

    z_q = _sc_gather(codebook.T, idx2.reshape(-1)).reshape(z.shape)
    return (z_q, indices, loss)


# speedup vs baseline: 1.0021x; 1.0021x over previous
import functools

import jax
import jax.numpy as jnp
from jax import lax
from jax.experimental import pallas as pl
from jax.experimental.pallas import tpu as pltpu
from jax.experimental.pallas import tpu_sc as plsc

_NUM_CODES = 8192
_CODE_DIM = 256
_BETA = 0.25

_ROW_BLOCK = 256
_GROUPS = ((0, 2816), (2816, 5632), (5632, 8192))


def _dist_argmin_body(z_ref, cb_ref, rn_ref, cn_ref, idx_ref, acc_ref,
                      cbb_ref):
    i = pl.program_id(0)

    @pl.when(i == 0)
    def _():
        cbb_ref[...] = cb_ref[...].astype(jnp.bfloat16)
    zm2 = (z_ref[...] * (-2.0)).astype(jnp.bfloat16)
    rn = rn_ref[...]
    lane = lax.broadcasted_iota(jnp.int32, (_ROW_BLOCK, 128), 1)

    run_cmp = None
    run_val = None
    run_idx = None
    for g, (c0, c1) in enumerate(_GROUPS):
        dotm = jnp.dot(zm2, cbb_ref[:, c0:c1],
                       preferred_element_type=jnp.float32)
        val = (rn + dotm[:, 0:128]) + cn_ref[:, c0:c0 + 128]
        kidx = jnp.zeros((_ROW_BLOCK, 128), jnp.int32)
        for k in range(1, (c1 - c0) // 128):
            dk = ((rn + dotm[:, k * 128:(k + 1) * 128])
                  + cn_ref[:, c0 + k * 128:c0 + (k + 1) * 128])
            hit = dk < val
            val = jnp.minimum(val, dk)
            kidx = jnp.where(hit, k, kidx)
        mv = jnp.min(val, axis=1, keepdims=True)
        gidx = kidx * 128 + lane + c0
        mi = jnp.min(jnp.where(val == mv, gidx, _NUM_CODES),
                     axis=1, keepdims=True)
        if g == 0:
            run_cmp, run_val, run_idx = mv, mv, mi
        else:
            better = mv < run_cmp
            run_cmp = jnp.where(better, mv, run_cmp)
            run_val = jnp.where(better, mv, run_val)
            run_idx = jnp.where(better, mi, run_idx)
        if g != len(_GROUPS) - 1:
            run_cmp = run_cmp.astype(jnp.bfloat16).astype(jnp.float32)

    idx_ref[...] = run_idx

    @pl.when(i == 0)
    def _():
        acc_ref[...] = jnp.zeros((1, 1), jnp.float32)

    acc_ref[...] += jnp.sum(run_val, keepdims=True)


def _sc_gather(table, idx):
    nc, ns = 2, 16
    nw = nc * ns
    b = idx.shape[0]
    d_ = table.shape[1]
    per_w = b // nw
    ch = 256
    n_ch = per_w // ch
    mesh = plsc.VectorSubcoreMesh(core_axis_name="c", subcore_axis_name="s")

    @functools.partial(
        pl.kernel, mesh=mesh,
        out_type=jax.ShapeDtypeStruct((b, d_), jnp.float32),
        scratch_types=[
            pltpu.VMEM((ch,), jnp.int32),
            pltpu.VMEM((ch, d_), jnp.float32),
            pltpu.SemaphoreType.DMA,
        ],
    )
    def k(table_hbm, idx_hbm, out_hbm, idx_v, rows_v, sem):
        wid = lax.axis_index("s") * nc + lax.axis_index("c")
        for c in range(n_ch):
            base = wid * per_w + c * ch
            pltpu.sync_copy(idx_hbm.at[pl.ds(base, ch)], idx_v)
            pltpu.async_copy(table_hbm.at[idx_v], rows_v, sem).wait()
            pltpu.sync_copy(rows_v, out_hbm.at[pl.ds(base, ch)])

    return k(table, idx)


def kernel(z, codebook):
    z_flat = z.reshape(-1, _CODE_DIM)
    n_rows = z_flat.shape[0]
    rn = jnp.sum(z_flat ** 2, axis=1, keepdims=True)
    cn = jnp.sum(codebook ** 2, axis=0, keepdims=True)

    grid = n_rows // _ROW_BLOCK
    idx2, acc = pl.pallas_call(
        _dist_argmin_body,
        grid=(grid,),
        in_specs=[
            pl.BlockSpec((_ROW_BLOCK, _CODE_DIM), lambda i: (i, 0)),
            pl.BlockSpec((_CODE_DIM, _NUM_CODES), lambda i: (0, 0)),
            pl.BlockSpec((_ROW_BLOCK, 1), lambda i: (i, 0)),
            pl.BlockSpec((1, _NUM_CODES), lambda i: (0, 0)),
        ],
        out_specs=[
            pl.BlockSpec((_ROW_BLOCK, 1), lambda i: (i, 0)),
            pl.BlockSpec((1, 1), lambda i: (0, 0)),
        ],
        out_shape=[
            jax.ShapeDtypeStruct((n_rows, 1), jnp.int32),
            jax.ShapeDtypeStruct((1, 1), jnp.float32),
        ],
        scratch_shapes=[pltpu.VMEM((_CODE_DIM, _NUM_CODES), jnp.bfloat16)],
    )(z_flat, codebook, rn, cn)

    indices = idx2.reshape(z.shape[:-1])
    loss = acc[0, 0] * ((1.0 + _BETA) / z.size)

    z_q = _sc_gather(codebook.T, idx2.reshape(-1)).reshape(z.shape)
    return (z_q, indices, loss)


# speedup vs baseline: 1.1303x; 1.1280x over previous
import functools

import jax
import jax.numpy as jnp
from jax import lax
from jax.experimental import pallas as pl
from jax.experimental.pallas import tpu as pltpu
from jax.experimental.pallas import tpu_sc as plsc

_NUM_CODES = 8192
_CODE_DIM = 256
_BETA = 0.25

_ROW_BLOCK = 512
_GROUPS = ((0, 2816), (2816, 5632), (5632, 8192))


def _dist_argmin_body(z_ref, cb_ref, rn_ref, cn_ref, idx_ref, acc_ref,
                      cbb_ref):
    i = pl.program_id(0)

    @pl.when(i == 0)
    def _():
        cbb_ref[...] = cb_ref[...].astype(jnp.bfloat16)
    zm2 = (z_ref[...] * (-2.0)).astype(jnp.bfloat16)
    rn = rn_ref[...]
    lane = lax.broadcasted_iota(jnp.int32, (_ROW_BLOCK, 128), 1)

    run_cmp = None
    run_val = None
    run_idx = None
    for g, (c0, c1) in enumerate(_GROUPS):
        dotm = jnp.dot(zm2, cbb_ref[:, c0:c1],
                       preferred_element_type=jnp.float32)
        val = (rn + dotm[:, 0:128]) + cn_ref[:, c0:c0 + 128]
        kidx = jnp.zeros((_ROW_BLOCK, 128), jnp.int32)
        for k in range(1, (c1 - c0) // 128):
            dk = ((rn + dotm[:, k * 128:(k + 1) * 128])
                  + cn_ref[:, c0 + k * 128:c0 + (k + 1) * 128])
            hit = dk < val
            val = jnp.minimum(val, dk)
            kidx = jnp.where(hit, k, kidx)
        mv = jnp.min(val, axis=1, keepdims=True)
        gidx = kidx * 128 + lane + c0
        mi = jnp.min(jnp.where(val == mv, gidx, _NUM_CODES),
                     axis=1, keepdims=True)
        if g == 0:
            run_cmp, run_val, run_idx = mv, mv, mi
        else:
            better = mv < run_cmp
            run_cmp = jnp.where(better, mv, run_cmp)
            run_val = jnp.where(better, mv, run_val)
            run_idx = jnp.where(better, mi, run_idx)
        if g != len(_GROUPS) - 1:
            run_cmp = run_cmp.astype(jnp.bfloat16).astype(jnp.float32)

    idx_ref[...] = run_idx

    @pl.when(i == 0)
    def _():
        acc_ref[...] = jnp.zeros((1, 1), jnp.float32)

    acc_ref[...] += jnp.sum(run_val, keepdims=True)


def _sc_gather(table, idx):
    nc, ns = 2, 16
    nw = nc * ns
    b = idx.shape[0]
    d_ = table.shape[1]
    per_w = b // nw
    ch = 256
    n_ch = per_w // ch
    mesh = plsc.VectorSubcoreMesh(core_axis_name="c", subcore_axis_name="s")

    @functools.partial(
        pl.kernel, mesh=mesh,
        out_type=jax.ShapeDtypeStruct((b, d_), jnp.float32),
        scratch_types=[
            pltpu.VMEM((ch,), jnp.int32),
            pltpu.VMEM((ch, d_), jnp.float32),
            pltpu.SemaphoreType.DMA,
        ],
    )
    def k(table_hbm, idx_hbm, out_hbm, idx_v, rows_v, sem):
        wid = lax.axis_index("s") * nc + lax.axis_index("c")
        for c in range(n_ch):
            base = wid * per_w + c * ch
            pltpu.sync_copy(idx_hbm.at[pl.ds(base, ch)], idx_v)
            pltpu.async_copy(table_hbm.at[idx_v], rows_v, sem).wait()
            pltpu.sync_copy(rows_v, out_hbm.at[pl.ds(base, ch)])

    return k(table, idx)


def kernel(z, codebook):
    z_flat = z.reshape(-1, _CODE_DIM)
    n_rows = z_flat.shape[0]
    rn = jnp.sum(z_flat ** 2, axis=1, keepdims=True)
    cn = jnp.sum(codebook ** 2, axis=0, keepdims=True)

    grid = n_rows // _ROW_BLOCK
    idx2, acc = pl.pallas_call(
        _dist_argmin_body,
        grid=(grid,),
        in_specs=[
            pl.BlockSpec((_ROW_BLOCK, _CODE_DIM), lambda i: (i, 0)),
            pl.BlockSpec((_CODE_DIM, _NUM_CODES), lambda i: (0, 0)),
            pl.BlockSpec((_ROW_BLOCK, 1), lambda i: (i, 0)),
            pl.BlockSpec((1, _NUM_CODES), lambda i: (0, 0)),
        ],
        out_specs=[
            pl.BlockSpec((_ROW_BLOCK, 1), lambda i: (i, 0)),
            pl.BlockSpec((1, 1), lambda i: (0, 0)),
        ],
        out_shape=[
            jax.ShapeDtypeStruct((n_rows, 1), jnp.int32),
            jax.ShapeDtypeStruct((1, 1), jnp.float32),
        ],
        scratch_shapes=[pltpu.VMEM((_CODE_DIM, _NUM_CODES), jnp.bfloat16)],
    )(z_flat, codebook, rn, cn)

    indices = idx2.reshape(z.shape[:-1])
    loss = acc[0, 0] * ((1.0 + _BETA) / z.size)

    z_q = _sc_gather(codebook.T, idx2.reshape(-1)).reshape(z.shape)
    return (z_q, indices, loss)


# speedup vs baseline: 1.1734x; 1.0381x over previous
import functools

import jax
import jax.numpy as jnp
from jax import lax
from jax.experimental import pallas as pl
from jax.experimental.pallas import tpu as pltpu
from jax.experimental.pallas import tpu_sc as plsc

_NUM_CODES = 8192
_CODE_DIM = 256
_BETA = 0.25

_ROW_BLOCK = 1024
_GROUPS = ((0, 2816), (2816, 5632), (5632, 8192))


def _dist_argmin_body(z_ref, cb_ref, rn_ref, cn_ref, idx_ref, acc_ref,
                      cbb_ref):
    i = pl.program_id(0)

    @pl.when(i == 0)
    def _():
        cbb_ref[...] = cb_ref[...].astype(jnp.bfloat16)
    zm2 = (z_ref[...] * (-2.0)).astype(jnp.bfloat16)
    rn = rn_ref[...]
    lane = lax.broadcasted_iota(jnp.int32, (_ROW_BLOCK, 128), 1)

    run_cmp = None
    run_val = None
    run_idx = None
    for g, (c0, c1) in enumerate(_GROUPS):
        dotm = jnp.dot(zm2, cbb_ref[:, c0:c1],
                       preferred_element_type=jnp.float32)
        val = (rn + dotm[:, 0:128]) + cn_ref[:, c0:c0 + 128]
        kidx = jnp.zeros((_ROW_BLOCK, 128), jnp.int32)
        for k in range(1, (c1 - c0) // 128):
            dk = ((rn + dotm[:, k * 128:(k + 1) * 128])
                  + cn_ref[:, c0 + k * 128:c0 + (k + 1) * 128])
            hit = dk < val
            val = jnp.minimum(val, dk)
            kidx = jnp.where(hit, k, kidx)
        mv = jnp.min(val, axis=1, keepdims=True)
        gidx = kidx * 128 + lane + c0
        mi = jnp.min(jnp.where(val == mv, gidx, _NUM_CODES),
                     axis=1, keepdims=True)
        if g == 0:
            run_cmp, run_val, run_idx = mv, mv, mi
        else:
            better = mv < run_cmp
            run_cmp = jnp.where(better, mv, run_cmp)
            run_val = jnp.where(better, mv, run_val)
            run_idx = jnp.where(better, mi, run_idx)
        if g != len(_GROUPS) - 1:
            run_cmp = run_cmp.astype(jnp.bfloat16).astype(jnp.float32)

    idx_ref[...] = run_idx

    @pl.when(i == 0)
    def _():
        acc_ref[...] = jnp.zeros((1, 1), jnp.float32)

    acc_ref[...] += jnp.sum(run_val, keepdims=True)


def _sc_gather(table, idx):
    nc, ns = 2, 16
    nw = nc * ns
    b = idx.shape[0]
    d_ = table.shape[1]
    per_w = b // nw
    ch = 256
    n_ch = per_w // ch
    mesh = plsc.VectorSubcoreMesh(core_axis_name="c", subcore_axis_name="s")

    @functools.partial(
        pl.kernel, mesh=mesh,
        out_type=jax.ShapeDtypeStruct((b, d_), jnp.float32),
        scratch_types=[
            pltpu.VMEM((ch,), jnp.int32),
            pltpu.VMEM((ch, d_), jnp.float32),
            pltpu.SemaphoreType.DMA,
        ],
    )
    def k(table_hbm, idx_hbm, out_hbm, idx_v, rows_v, sem):
        wid = lax.axis_index("s") * nc + lax.axis_index("c")
        for c in range(n_ch):
            base = wid * per_w + c * ch
            pltpu.sync_copy(idx_hbm.at[pl.ds(base, ch)], idx_v)
            pltpu.async_copy(table_hbm.at[idx_v], rows_v, sem).wait()
            pltpu.sync_copy(rows_v, out_hbm.at[pl.ds(base, ch)])

    return k(table, idx)


def kernel(z, codebook):
    z_flat = z.reshape(-1, _CODE_DIM)
    n_rows = z_flat.shape[0]
    rn = jnp.sum(z_flat ** 2, axis=1, keepdims=True)
    cn = jnp.sum(codebook ** 2, axis=0, keepdims=True)

    grid = n_rows // _ROW_BLOCK
    idx2, acc = pl.pallas_call(
        _dist_argmin_body,
        grid=(grid,),
        in_specs=[
            pl.BlockSpec((_ROW_BLOCK, _CODE_DIM), lambda i: (i, 0)),
            pl.BlockSpec((_CODE_DIM, _NUM_CODES), lambda i: (0, 0)),
            pl.BlockSpec((_ROW_BLOCK, 1), lambda i: (i, 0)),
            pl.BlockSpec((1, _NUM_CODES), lambda i: (0, 0)),
        ],
        out_specs=[
            pl.BlockSpec((_ROW_BLOCK, 1), lambda i: (i, 0)),
            pl.BlockSpec((1, 1), lambda i: (0, 0)),
        ],
        out_shape=[
            jax.ShapeDtypeStruct((n_rows, 1), jnp.int32),
            jax.ShapeDtypeStruct((1, 1), jnp.float32),
        ],
        scratch_shapes=[pltpu.VMEM((_CODE_DIM, _NUM_CODES), jnp.bfloat16)],
    )(z_flat, codebook, rn, cn)

    indices = idx2.reshape(z.shape[:-1])
    loss = acc[0, 0] * ((1.0 + _BETA) / z.size)

    z_q = _sc_gather(codebook.T, idx2.reshape(-1)).reshape(z.shape)
    return (z_q, indices, loss)


# speedup vs baseline: 1.2296x; 1.0479x over previous
import functools

import jax
import jax.numpy as jnp
from jax import lax
from jax.experimental import pallas as pl
from jax.experimental.pallas import tpu as pltpu
from jax.experimental.pallas import tpu_sc as plsc

_NUM_CODES = 8192
_CODE_DIM = 256
_BETA = 0.25

_ROW_BLOCK = 2048
_GROUPS = ((0, 2816), (2816, 5632), (5632, 8192))


def _dist_argmin_body(z_ref, cb_ref, rn_ref, cn_ref, idx_ref, acc_ref,
                      cbb_ref):
    i = pl.program_id(0)

    @pl.when(i == 0)
    def _():
        cbb_ref[...] = cb_ref[...].astype(jnp.bfloat16)
    zm2 = (z_ref[...] * (-2.0)).astype(jnp.bfloat16)
    rn = rn_ref[...]
    lane = lax.broadcasted_iota(jnp.int32, (_ROW_BLOCK, 128), 1)

    run_cmp = None
    run_val = None
    run_idx = None
    for g, (c0, c1) in enumerate(_GROUPS):
        dotm = jnp.dot(zm2, cbb_ref[:, c0:c1],
                       preferred_element_type=jnp.float32)
        val = (rn + dotm[:, 0:128]) + cn_ref[:, c0:c0 + 128]
        kidx = jnp.zeros((_ROW_BLOCK, 128), jnp.int32)
        for k in range(1, (c1 - c0) // 128):
            dk = ((rn + dotm[:, k * 128:(k + 1) * 128])
                  + cn_ref[:, c0 + k * 128:c0 + (k + 1) * 128])
            hit = dk < val
            val = jnp.minimum(val, dk)
            kidx = jnp.where(hit, k, kidx)
        mv = jnp.min(val, axis=1, keepdims=True)
        gidx = kidx * 128 + lane + c0
        mi = jnp.min(jnp.where(val == mv, gidx, _NUM_CODES),
                     axis=1, keepdims=True)
        if g == 0:
            run_cmp, run_val, run_idx = mv, mv, mi
        else:
            better = mv < run_cmp
            run_cmp = jnp.where(better, mv, run_cmp)
            run_val = jnp.where(better, mv, run_val)
            run_idx = jnp.where(better, mi, run_idx)
        if g != len(_GROUPS) - 1:
            run_cmp = run_cmp.astype(jnp.bfloat16).astype(jnp.float32)

    idx_ref[...] = run_idx

    @pl.when(i == 0)
    def _():
        acc_ref[...] = jnp.zeros((1, 1), jnp.float32)

    acc_ref[...] += jnp.sum(run_val, keepdims=True)


def _sc_gather(table, idx):
    nc, ns = 2, 16
    nw = nc * ns
    b = idx.shape[0]
    d_ = table.shape[1]
    per_w = b // nw
    ch = 256
    n_ch = per_w // ch
    mesh = plsc.VectorSubcoreMesh(core_axis_name="c", subcore_axis_name="s")

    @functools.partial(
        pl.kernel, mesh=mesh,
        out_type=jax.ShapeDtypeStruct((b, d_), jnp.float32),
        scratch_types=[
            pltpu.VMEM((ch,), jnp.int32),
            pltpu.VMEM((ch, d_), jnp.float32),
            pltpu.SemaphoreType.DMA,
        ],
    )
    def k(table_hbm, idx_hbm, out_hbm, idx_v, rows_v, sem):
        wid = lax.axis_index("s") * nc + lax.axis_index("c")
        for c in range(n_ch):
            base = wid * per_w + c * ch
            pltpu.sync_copy(idx_hbm.at[pl.ds(base, ch)], idx_v)
            pltpu.async_copy(table_hbm.at[idx_v], rows_v, sem).wait()
            pltpu.sync_copy(rows_v, out_hbm.at[pl.ds(base, ch)])

    return k(table, idx)


def kernel(z, codebook):
    z_flat = z.reshape(-1, _CODE_DIM)
    n_rows = z_flat.shape[0]
    rn = jnp.sum(z_flat ** 2, axis=1, keepdims=True)
    cn = jnp.sum(codebook ** 2, axis=0, keepdims=True)

    grid = n_rows // _ROW_BLOCK
    idx2, acc = pl.pallas_call(
        _dist_argmin_body,
        grid=(grid,),
        in_specs=[
            pl.BlockSpec((_ROW_BLOCK, _CODE_DIM), lambda i: (i, 0)),
            pl.BlockSpec((_CODE_DIM, _NUM_CODES), lambda i: (0, 0)),
            pl.BlockSpec((_ROW_BLOCK, 1), lambda i: (i, 0)),
            pl.BlockSpec((1, _NUM_CODES), lambda i: (0, 0)),
        ],
        out_specs=[
            pl.BlockSpec((_ROW_BLOCK, 1), lambda i: (i, 0)),
            pl.BlockSpec((1, 1), lambda i: (0, 0)),
        ],
        out_shape=[
            jax.ShapeDtypeStruct((n_rows, 1), jnp.int32),
            jax.ShapeDtypeStruct((1, 1), jnp.float32),
        ],
        scratch_shapes=[pltpu.VMEM((_CODE_DIM, _NUM_CODES), jnp.bfloat16)],
    )(z_flat, codebook, rn, cn)

    indices = idx2.reshape(z.shape[:-1])
    loss = acc[0, 0] * ((1.0 + _BETA) / z.size)

    z_q = _sc_gather(codebook.T, idx2.reshape(-1)).reshape(z.shape)
    return (z_q, indices, loss)


# speedup vs baseline: 1.2710x; 1.0337x over previous
import functools

import jax
import jax.numpy as jnp
from jax import lax
from jax.experimental import pallas as pl
from jax.experimental.pallas import tpu as pltpu
from jax.experimental.pallas import tpu_sc as plsc

_NUM_CODES = 8192
_CODE_DIM = 256
_BETA = 0.25

_ROW_BLOCK = 4096
_GROUPS = ((0, 2816), (2816, 5632), (5632, 8192))


def _dist_argmin_body(z_ref, cb_ref, rn_ref, cn_ref, idx_ref, acc_ref,
                      cbb_ref):
    i = pl.program_id(0)

    @pl.when(i == 0)
    def _():
        cbb_ref[...] = cb_ref[...].astype(jnp.bfloat16)
    zm2 = (z_ref[...] * (-2.0)).astype(jnp.bfloat16)
    rn = rn_ref[...]
    lane = lax.broadcasted_iota(jnp.int32, (_ROW_BLOCK, 128), 1)

    run_cmp = None
    run_val = None
    run_idx = None
    for g, (c0, c1) in enumerate(_GROUPS):
        dotm = jnp.dot(zm2, cbb_ref[:, c0:c1],
                       preferred_element_type=jnp.float32)
        val = (rn + dotm[:, 0:128]) + cn_ref[:, c0:c0 + 128]
        kidx = jnp.zeros((_ROW_BLOCK, 128), jnp.int32)
        for k in range(1, (c1 - c0) // 128):
            dk = ((rn + dotm[:, k * 128:(k + 1) * 128])
                  + cn_ref[:, c0 + k * 128:c0 + (k + 1) * 128])
            hit = dk < val
            val = jnp.minimum(val, dk)
            kidx = jnp.where(hit, k, kidx)
        mv = jnp.min(val, axis=1, keepdims=True)
        gidx = kidx * 128 + lane + c0
        mi = jnp.min(jnp.where(val == mv, gidx, _NUM_CODES),
                     axis=1, keepdims=True)
        if g == 0:
            run_cmp, run_val, run_idx = mv, mv, mi
        else:
            better = mv < run_cmp
            run_cmp = jnp.where(better, mv, run_cmp)
            run_val = jnp.where(better, mv, run_val)
            run_idx = jnp.where(better, mi, run_idx)
        if g != len(_GROUPS) - 1:
            run_cmp = run_cmp.astype(jnp.bfloat16).astype(jnp.float32)

    idx_ref[...] = run_idx

    @pl.when(i == 0)
    def _():
        acc_ref[...] = jnp.zeros((1, 1), jnp.float32)

    acc_ref[...] += jnp.sum(run_val, keepdims=True)


def _sc_gather(table, idx):
    nc, ns = 2, 16
    nw = nc * ns
    b = idx.shape[0]
    d_ = table.shape[1]
    per_w = b // nw
    ch = 256
    n_ch = per_w // ch
    mesh = plsc.VectorSubcoreMesh(core_axis_name="c", subcore_axis_name="s")

    @functools.partial(
        pl.kernel, mesh=mesh,
        out_type=jax.ShapeDtypeStruct((b, d_), jnp.float32),
        scratch_types=[
            pltpu.VMEM((ch,), jnp.int32),
            pltpu.VMEM((ch, d_), jnp.float32),
            pltpu.SemaphoreType.DMA,
        ],
    )
    def k(table_hbm, idx_hbm, out_hbm, idx_v, rows_v, sem):
        wid = lax.axis_index("s") * nc + lax.axis_index("c")
        for c in range(n_ch):
            base = wid * per_w + c * ch
            pltpu.sync_copy(idx_hbm.at[pl.ds(base, ch)], idx_v)
            pltpu.async_copy(table_hbm.at[idx_v], rows_v, sem).wait()
            pltpu.sync_copy(rows_v, out_hbm.at[pl.ds(base, ch)])

    return k(table, idx)


def kernel(z, codebook):
    z_flat = z.reshape(-1, _CODE_DIM)
    n_rows = z_flat.shape[0]
    rn = jnp.sum(z_flat ** 2, axis=1, keepdims=True)
    cn = jnp.sum(codebook ** 2, axis=0, keepdims=True)

    grid = n_rows // _ROW_BLOCK
    idx2, acc = pl.pallas_call(
        _dist_argmin_body,
        grid=(grid,),
        in_specs=[
            pl.BlockSpec((_ROW_BLOCK, _CODE_DIM), lambda i: (i, 0)),
            pl.BlockSpec((_CODE_DIM, _NUM_CODES), lambda i: (0, 0)),
            pl.BlockSpec((_ROW_BLOCK, 1), lambda i: (i, 0)),
            pl.BlockSpec((1, _NUM_CODES), lambda i: (0, 0)),
        ],
        out_specs=[
            pl.BlockSpec((_ROW_BLOCK, 1), lambda i: (i, 0)),
            pl.BlockSpec((1, 1), lambda i: (0, 0)),
        ],
        out_shape=[
            jax.ShapeDtypeStruct((n_rows, 1), jnp.int32),
            jax.ShapeDtypeStruct((1, 1), jnp.float32),
        ],
        scratch_shapes=[pltpu.VMEM((_CODE_DIM, _NUM_CODES), jnp.bfloat16)],
    )(z_flat, codebook, rn, cn)

    indices = idx2.reshape(z.shape[:-1])
    loss = acc[0, 0] * ((1.0 + _BETA) / z.size)

    z_q = _sc_gather(codebook.T, idx2.reshape(-1)).reshape(z.shape)
    return (z_q, indices, loss)


# speedup vs baseline: 1.3414x; 1.0553x over previous
import functools

import jax
import jax.numpy as jnp
from jax import lax
from jax.experimental import pallas as pl
from jax.experimental.pallas import tpu as pltpu
from jax.experimental.pallas import tpu_sc as plsc

_NUM_CODES = 8192
_CODE_DIM = 256
_BETA = 0.25

_ROW_BLOCK = 4096
_GROUPS = ((0, 2816), (2816, 5632), (5632, 8192))


def _dist_argmin_body(z_ref, cb_ref, cn_ref, idx_ref, acc_ref,
                      cbb_ref):
    i = pl.program_id(0)

    @pl.when(i == 0)
    def _():
        cbb_ref[...] = cb_ref[...].astype(jnp.bfloat16)
    zf = z_ref[...]
    zm2 = (zf * (-2.0)).astype(jnp.bfloat16)
    rn = jnp.sum(zf * zf, axis=1, keepdims=True)
    lane = lax.broadcasted_iota(jnp.int32, (_ROW_BLOCK, 128), 1)

    run_cmp = None
    run_val = None
    run_idx = None
    for g, (c0, c1) in enumerate(_GROUPS):
        dotm = jnp.dot(zm2, cbb_ref[:, c0:c1],
                       preferred_element_type=jnp.float32)
        val = (rn + dotm[:, 0:128]) + cn_ref[:, c0:c0 + 128]
        kidx = jnp.zeros((_ROW_BLOCK, 128), jnp.int32)
        for k in range(1, (c1 - c0) // 128):
            dk = ((rn + dotm[:, k * 128:(k + 1) * 128])
                  + cn_ref[:, c0 + k * 128:c0 + (k + 1) * 128])
            hit = dk < val
            val = jnp.minimum(val, dk)
            kidx = jnp.where(hit, k, kidx)
        mv = jnp.min(val, axis=1, keepdims=True)
        gidx = kidx * 128 + lane + c0
        mi = jnp.min(jnp.where(val == mv, gidx, _NUM_CODES),
                     axis=1, keepdims=True)
        if g == 0:
            run_cmp, run_val, run_idx = mv, mv, mi
        else:
            better = mv < run_cmp
            run_cmp = jnp.where(better, mv, run_cmp)
            run_val = jnp.where(better, mv, run_val)
            run_idx = jnp.where(better, mi, run_idx)
        if g != len(_GROUPS) - 1:
            run_cmp = run_cmp.astype(jnp.bfloat16).astype(jnp.float32)

    idx_ref[...] = run_idx

    @pl.when(i == 0)
    def _():
        acc_ref[...] = jnp.zeros((1, 1), jnp.float32)

    acc_ref[...] += jnp.sum(run_val, keepdims=True)


def _sc_gather(table, idx):
    nc, ns = 2, 16
    nw = nc * ns
    b = idx.shape[0]
    d_ = table.shape[1]
    per_w = b // nw
    ch = 256
    n_ch = per_w // ch
    mesh = plsc.VectorSubcoreMesh(core_axis_name="c", subcore_axis_name="s")

    @functools.partial(
        pl.kernel, mesh=mesh,
        out_type=jax.ShapeDtypeStruct((b, d_), jnp.float32),
        scratch_types=[
            pltpu.VMEM((ch,), jnp.int32),
            pltpu.VMEM((ch, d_), jnp.float32),
            pltpu.SemaphoreType.DMA,
        ],
    )
    def k(table_hbm, idx_hbm, out_hbm, idx_v, rows_v, sem):
        wid = lax.axis_index("s") * nc + lax.axis_index("c")
        for c in range(n_ch):
            base = wid * per_w + c * ch
            pltpu.sync_copy(idx_hbm.at[pl.ds(base, ch)], idx_v)
            pltpu.async_copy(table_hbm.at[idx_v], rows_v, sem).wait()
            pltpu.sync_copy(rows_v, out_hbm.at[pl.ds(base, ch)])

    return k(table, idx)


def kernel(z, codebook):
    z_flat = z.reshape(-1, _CODE_DIM)
    n_rows = z_flat.shape[0]
    cn = jnp.sum(codebook ** 2, axis=0, keepdims=True)

    grid = n_rows // _ROW_BLOCK
    idx2, acc = pl.pallas_call(
        _dist_argmin_body,
        grid=(grid,),
        in_specs=[
            pl.BlockSpec((_ROW_BLOCK, _CODE_DIM), lambda i: (i, 0)),
            pl.BlockSpec((_CODE_DIM, _NUM_CODES), lambda i: (0, 0)),
            pl.BlockSpec((1, _NUM_CODES), lambda i: (0, 0)),
        ],
        out_specs=[
            pl.BlockSpec((_ROW_BLOCK, 1), lambda i: (i, 0)),
            pl.BlockSpec((1, 1), lambda i: (0, 0)),
        ],
        out_shape=[
            jax.ShapeDtypeStruct((n_rows, 1), jnp.int32),
            jax.ShapeDtypeStruct((1, 1), jnp.float32),
        ],
        scratch_shapes=[pltpu.VMEM((_CODE_DIM, _NUM_CODES), jnp.bfloat16)],
    )(z_flat, codebook, cn)

    indices = idx2.reshape(z.shape[:-1])
    loss = acc[0, 0] * ((1.0 + _BETA) / z.size)

    z_q = _sc_gather(codebook.T, idx2.reshape(-1)).reshape(z.shape)
    return (z_q, indices, loss)
